# Initial kernel scaffold; baseline (speedup 1.0000x reference)
import jax
import jax.numpy as jnp
from jax import lax
from jax.experimental import pallas as pl
from jax.experimental.pallas import tpu as pltpu
from jax.experimental.pallas import tpu_sc as plsc

_NC = 2
_NS = 16
_NW = _NC * _NS
_CH = 128
_NBUF = 4
_DEGW = 16


def _make_agg(n_acc, n_chunk, d, with_deg):
  mesh = plsc.VectorSubcoreMesh(core_axis_name="c", subcore_axis_name="s")
  rows_per_tile = n_acc // _NS
  out_type = [jax.ShapeDtypeStruct((_NC, n_acc, d), jnp.float32)]
  if with_deg:
    out_type.append(jax.ShapeDtypeStruct((_NC, n_acc, _DEGW), jnp.float32))

  scratch = [
      pltpu.VMEM_SHARED((n_acc, d), jnp.float32),
      pltpu.VMEM((n_chunk, _CH), jnp.int32),
      pltpu.VMEM((n_chunk, _CH), jnp.int32),
      pltpu.VMEM((_NBUF, _CH, d), jnp.float32),
      pltpu.VMEM((_CH, d), jnp.float32),
      pltpu.SemaphoreType.DMA((_NBUF,)),
  ]
  if with_deg:
    scratch += [
        pltpu.VMEM_SHARED((n_acc, _DEGW), jnp.float32),
        pltpu.VMEM((rows_per_tile, _DEGW), jnp.float32),
        pltpu.VMEM((_CH, _DEGW), jnp.float32),
    ]

  def body(g_h, src_h, dst_h, *rest):
    if with_deg:
      (part_h, degp_h, acc, src_v, dst_v, rows, zb, sems,
       deg_sh, zdeg, ones_v) = rest
    else:
      part_h, acc, src_v, dst_v, rows, zb, sems = rest
    c = lax.axis_index("c")
    s = lax.axis_index("s")
    w = c * _NS + s

    pltpu.sync_copy(src_h.at[w], src_v)
    pltpu.sync_copy(dst_h.at[w], dst_v)

    zv = jnp.zeros((16,), jnp.float32)
    ncol = d // 16

    def zrow(i, _):
      zb[i // ncol, pl.ds(lax.rem(i, ncol) * 16, 16)] = zv
      return 0
    lax.fori_loop(0, _CH * ncol, zrow, 0)
    if with_deg:
      def zdrow(i, _):
        zdeg[i, :] = zv
        return 0
      lax.fori_loop(0, rows_per_tile, zdrow, 0)

      ov = jnp.ones((16,), jnp.float32)
      def orow(i, _):
        ones_v[i, :] = ov
        return 0
      lax.fori_loop(0, _CH, orow, 0)

    base = s * rows_per_tile
    nfull = rows_per_tile // _CH
    for j in range(nfull):
      pltpu.sync_copy(zb, acc.at[pl.ds(base + j * _CH, _CH)])
    rem = rows_per_tile - nfull * _CH
    if rem:
      pltpu.sync_copy(zb.at[pl.ds(0, rem)],
                      acc.at[pl.ds(base + nfull * _CH, rem)])
    if with_deg:
      pltpu.sync_copy(zdeg, deg_sh.at[pl.ds(base, rows_per_tile)])
    plsc.subcore_barrier()

    for b in range(_NBUF):
      pltpu.async_copy(g_h.at[src_v.at[b]], rows.at[b], sems.at[b])

    def step(k, _):
      b = lax.rem(k, _NBUF)
      pltpu.make_async_copy(g_h.at[src_v.at[k]], rows.at[b],
                            sems.at[b]).wait()
      pltpu.sync_copy(rows.at[b], acc.at[dst_v.at[k]], add=True)
      if with_deg:
        pltpu.sync_copy(ones_v, deg_sh.at[dst_v.at[k]], add=True)

      @pl.when(k + _NBUF < n_chunk)
      def _():
        pltpu.async_copy(g_h.at[src_v.at[k + _NBUF]], rows.at[b],
                         sems.at[b])
      return 0
    lax.fori_loop(0, n_chunk, step, 0)

    plsc.subcore_barrier()
    sl = pl.ds(base, rows_per_tile)
    pltpu.sync_copy(acc.at[sl], part_h.at[c, sl])
    if with_deg:
      pltpu.sync_copy(deg_sh.at[sl], degp_h.at[c, sl])

  return pl.kernel(body, out_type=tuple(out_type) if with_deg else out_type[0],
                   mesh=mesh, scratch_types=scratch)


def _mm(x, w):
  n, d = x.shape
  blk = 1000

  def body(x_ref, w_ref, o_ref):
    o_ref[...] = jnp.dot(x_ref[...], w_ref[...],
                         preferred_element_type=jnp.float32)

  return pl.pallas_call(
      body,
      grid=(n // blk,),
      in_specs=[pl.BlockSpec((blk, d), lambda i: (i, 0)),
                pl.BlockSpec((d, d), lambda i: (0, 0))],
      out_specs=pl.BlockSpec((blk, d), lambda i: (i, 0)),
      out_shape=jax.ShapeDtypeStruct((n, d), jnp.float32),
  )(x, w)


def _combine_mm(p, dp, b, w, n):
  d = p.shape[-1]
  blk = 1000

  def body(p_ref, dp_ref, b_ref, w_ref, o_ref):
    pa = p_ref[0] + p_ref[1]
    dg = dp_ref[0, :, 0:1] + dp_ref[1, :, 0:1]
    dinv = 1.0 / jnp.maximum(dg, 1.0)
    h = jnp.maximum(pa * dinv + b_ref[...], 0.0)
    o_ref[...] = jnp.dot(h, w_ref[...], preferred_element_type=jnp.float32)

  return pl.pallas_call(
      body,
      grid=(n // blk,),
      in_specs=[pl.BlockSpec((2, blk, d), lambda i: (0, i, 0)),
                pl.BlockSpec((2, blk, _DEGW), lambda i: (0, i, 0)),
                pl.BlockSpec((1, d), lambda i: (0, 0)),
                pl.BlockSpec((d, d), lambda i: (0, 0))],
      out_specs=pl.BlockSpec((blk, d), lambda i: (i, 0)),
      out_shape=jax.ShapeDtypeStruct((n, d), jnp.float32),
  )(p, dp, b, w)


def _final(q, dp, b, n):
  d = q.shape[-1]
  blk = 1000

  def body(q_ref, dp_ref, b_ref, o_ref):
    qa = q_ref[0] + q_ref[1]
    dg = dp_ref[0, :, 0:1] + dp_ref[1, :, 0:1]
    dinv = 1.0 / jnp.maximum(dg, 1.0)
    o_ref[...] = qa * dinv + b_ref[...]

  return pl.pallas_call(
      body,
      grid=(n // blk,),
      in_specs=[pl.BlockSpec((2, blk, d), lambda i: (0, i, 0)),
                pl.BlockSpec((2, blk, _DEGW), lambda i: (0, i, 0)),
                pl.BlockSpec((1, d), lambda i: (0, 0))],
      out_specs=pl.BlockSpec((blk, d), lambda i: (i, 0)),
      out_shape=jax.ShapeDtypeStruct((n, d), jnp.float32),
  )(q, dp, b)


def kernel(x, edge_index, W1, b1, W2, b2):
  n, d = x.shape
  e = edge_index.shape[1]
  n_acc = -(-(n + 128) // _NS) * _NS
  n_chunk = -(-e // (_NW * _CH))
  e_pad = n_chunk * _NW * _CH

  src = edge_index[0]
  dst = edge_index[1]
  pad = e_pad - e
  if pad:
    ar = jnp.arange(pad, dtype=jnp.int32)
    src = jnp.concatenate([src, (ar * 131) % n])
    dst = jnp.concatenate([dst, n + lax.rem(ar, 128)])
  src_t = src.reshape(_NW, n_chunk, _CH)
  dst_t = dst.reshape(_NW, n_chunk, _CH)

  agg1 = _make_agg(n_acc, n_chunk, d, True)
  agg2 = _make_agg(n_acc, n_chunk, d, False)

  g1 = _mm(x, W1)
  p, dp = agg1(g1, src_t, dst_t)
  g2 = _combine_mm(p, dp, b1.reshape(1, d), W2, n)
  q = agg2(g2, src_t, dst_t)
  return _final(q, dp, b2.reshape(1, d), n)


# speedup vs baseline: 10.8174x; 10.8174x over previous
import jax
import jax.numpy as jnp
from jax import lax
from jax.experimental import pallas as pl
from jax.experimental.pallas import tpu as pltpu
from jax.experimental.pallas import tpu_sc as plsc

_NC = 2
_NS = 16
_NW = _NC * _NS
_CH = 128
_NBUF = 2
_DEGW = 128


_NIDX = 4


def _make_agg(n_acc, n_chunk, d):
  mesh = plsc.VectorSubcoreMesh(core_axis_name="c", subcore_axis_name="s")
  rows_per_tile = n_acc // _NS
  out_type = jax.ShapeDtypeStruct((_NC, n_acc, d), jnp.float32)

  scratch = [
      pltpu.VMEM_SHARED((n_acc, d), jnp.float32),
      pltpu.VMEM((_NBUF, _CH, d), jnp.float32),
      pltpu.VMEM((_NIDX, 2, _CH), jnp.int32),
      pltpu.SemaphoreType.DMA((_NBUF,)),
      pltpu.SemaphoreType.DMA((_NIDX,)),
  ]

  def body(g_h, idx_h, part_h, acc, rows, islot, gsem, isem):
    c = lax.axis_index("c")
    s = lax.axis_index("s")
    w = c * _NS + s

    zv = jnp.zeros((16,), jnp.float32)
    ncol = d // 16

    def zrow(i, _):
      rows[0, i // ncol, pl.ds(lax.rem(i, ncol) * 16, 16)] = zv
      return 0
    lax.fori_loop(0, _CH * ncol, zrow, 0)

    base = s * rows_per_tile
    for j in range(rows_per_tile // _CH):
      pltpu.sync_copy(rows.at[0], acc.at[pl.ds(base + j * _CH, _CH)])
    plsc.subcore_barrier()

    def stage_idx(k):
      pltpu.async_copy(idx_h.at[w, lax.rem(k, n_chunk)],
                       islot.at[lax.rem(k, _NIDX)],
                       isem.at[lax.rem(k, _NIDX)])

    def wait_idx(k):
      pltpu.make_async_copy(idx_h.at[w, lax.rem(k, n_chunk)],
                            islot.at[lax.rem(k, _NIDX)],
                            isem.at[lax.rem(k, _NIDX)]).wait()

    def issue_gather(k):
      pltpu.async_copy(g_h.at[islot.at[lax.rem(k, _NIDX), 0]],
                       rows.at[lax.rem(k, _NBUF)],
                       gsem.at[lax.rem(k, _NBUF)])

    def wait_gather(k):
      pltpu.make_async_copy(g_h.at[islot.at[lax.rem(k, _NIDX), 0]],
                            rows.at[lax.rem(k, _NBUF)],
                            gsem.at[lax.rem(k, _NBUF)]).wait()

    for j in range(min(_NIDX, n_chunk)):
      stage_idx(j)
    for j in range(min(_NBUF, n_chunk)):
      wait_idx(j)
      issue_gather(j)

    def step(k, _):
      wait_gather(k)
      dsti = islot.at[lax.rem(k, _NIDX), 1]
      pltpu.sync_copy(rows.at[lax.rem(k, _NBUF)], acc.at[dsti], add=True)

      @pl.when(k + _NBUF < n_chunk)
      def _():
        wait_idx(k + _NBUF)
        issue_gather(k + _NBUF)

      @pl.when(k + _NIDX < n_chunk)
      def _():
        stage_idx(k + _NIDX)
      return 0
    lax.fori_loop(0, n_chunk, step, 0)

    plsc.subcore_barrier()
    sl = pl.ds(base, rows_per_tile)
    pltpu.sync_copy(acc.at[sl], part_h.at[c, sl])

  return pl.kernel(body, out_type=out_type, mesh=mesh, scratch_types=scratch)


def _make_deg(n_acc, n_chunk):
  mesh = plsc.VectorSubcoreMesh(core_axis_name="c", subcore_axis_name="s")
  rows_per_tile = n_acc // _NS
  out_type = jax.ShapeDtypeStruct((_NC, n_acc, _DEGW), jnp.float32)

  scratch = [
      pltpu.VMEM_SHARED((n_acc, _DEGW), jnp.float32),
      pltpu.VMEM((_CH, _DEGW), jnp.float32),
      pltpu.VMEM((_CH, _DEGW), jnp.float32),
      pltpu.VMEM((_NIDX, 2, _CH), jnp.int32),
      pltpu.SemaphoreType.DMA((_NIDX,)),
  ]

  def body(idx_h, degp_h, deg_sh, zdeg, ones_v, islot, isem):
    c = lax.axis_index("c")
    s = lax.axis_index("s")
    w = c * _NS + s

    zv = jnp.zeros((16,), jnp.float32)
    ov = jnp.ones((16,), jnp.float32)
    ncol = _DEGW // 16

    def frow(i, _):
      r = i // ncol
      q = pl.ds(lax.rem(i, ncol) * 16, 16)
      zdeg[r, q] = zv
      ones_v[r, q] = ov
      return 0
    lax.fori_loop(0, _CH * ncol, frow, 0)

    base = s * rows_per_tile
    for j in range(rows_per_tile // _CH):
      pltpu.sync_copy(zdeg, deg_sh.at[pl.ds(base + j * _CH, _CH)])
    plsc.subcore_barrier()

    def stage_idx(k):
      pltpu.async_copy(idx_h.at[w, lax.rem(k, n_chunk)],
                       islot.at[lax.rem(k, _NIDX)],
                       isem.at[lax.rem(k, _NIDX)])

    def wait_idx(k):
      pltpu.make_async_copy(idx_h.at[w, lax.rem(k, n_chunk)],
                            islot.at[lax.rem(k, _NIDX)],
                            isem.at[lax.rem(k, _NIDX)]).wait()

    for j in range(min(_NIDX, n_chunk)):
      stage_idx(j)

    def step(k, _):
      wait_idx(k)
      pltpu.sync_copy(ones_v, deg_sh.at[islot.at[lax.rem(k, _NIDX), 1]],
                      add=True)

      @pl.when(k + _NIDX < n_chunk)
      def _():
        stage_idx(k + _NIDX)
      return 0
    lax.fori_loop(0, n_chunk, step, 0)

    plsc.subcore_barrier()
    sl = pl.ds(base, rows_per_tile)
    pltpu.sync_copy(deg_sh.at[sl], degp_h.at[c, sl])

  return pl.kernel(body, out_type=out_type, mesh=mesh, scratch_types=scratch)


def _mm(x, w):
  n, d = x.shape
  blk = 1000

  def body(x_ref, w_ref, o_ref):
    o_ref[...] = jnp.dot(x_ref[...], w_ref[...],
                         preferred_element_type=jnp.float32)

  return pl.pallas_call(
      body,
      grid=(n // blk,),
      in_specs=[pl.BlockSpec((blk, d), lambda i: (i, 0)),
                pl.BlockSpec((d, d), lambda i: (0, 0))],
      out_specs=pl.BlockSpec((blk, d), lambda i: (i, 0)),
      out_shape=jax.ShapeDtypeStruct((n, d), jnp.float32),
  )(x, w)


def _combine_mm(p, dp, b, w, n):
  d = p.shape[-1]
  blk = 1000

  def body(p_ref, dp_ref, b_ref, w_ref, o_ref):
    pa = p_ref[0] + p_ref[1]
    dg = dp_ref[0, :, 0:1] + dp_ref[1, :, 0:1]
    dinv = 1.0 / jnp.maximum(dg, 1.0)
    h = jnp.maximum(pa * dinv + b_ref[...], 0.0)
    o_ref[...] = jnp.dot(h, w_ref[...], preferred_element_type=jnp.float32)

  return pl.pallas_call(
      body,
      grid=(n // blk,),
      in_specs=[pl.BlockSpec((2, blk, d), lambda i: (0, i, 0)),
                pl.BlockSpec((2, blk, _DEGW), lambda i: (0, i, 0)),
                pl.BlockSpec((1, d), lambda i: (0, 0)),
                pl.BlockSpec((d, d), lambda i: (0, 0))],
      out_specs=pl.BlockSpec((blk, d), lambda i: (i, 0)),
      out_shape=jax.ShapeDtypeStruct((n, d), jnp.float32),
  )(p, dp, b, w)


def _final(q, dp, b, n):
  d = q.shape[-1]
  blk = 1000

  def body(q_ref, dp_ref, b_ref, o_ref):
    qa = q_ref[0] + q_ref[1]
    dg = dp_ref[0, :, 0:1] + dp_ref[1, :, 0:1]
    dinv = 1.0 / jnp.maximum(dg, 1.0)
    o_ref[...] = qa * dinv + b_ref[...]

  return pl.pallas_call(
      body,
      grid=(n // blk,),
      in_specs=[pl.BlockSpec((2, blk, d), lambda i: (0, i, 0)),
                pl.BlockSpec((2, blk, _DEGW), lambda i: (0, i, 0)),
                pl.BlockSpec((1, d), lambda i: (0, 0))],
      out_specs=pl.BlockSpec((blk, d), lambda i: (i, 0)),
      out_shape=jax.ShapeDtypeStruct((n, d), jnp.float32),
  )(q, dp, b)


def kernel(x, edge_index, W1, b1, W2, b2):
  n, d = x.shape
  e = edge_index.shape[1]
  n_acc = -(-(n + 128) // (_NS * 8)) * (_NS * 8)
  n_chunk = -(-e // (_NW * _CH))
  e_pad = n_chunk * _NW * _CH

  src = edge_index[0]
  dst = edge_index[1]
  pad = e_pad - e
  if pad:
    ar = jnp.arange(pad, dtype=jnp.int32)
    src = jnp.concatenate([src, (ar * 131) % n])
    dst = jnp.concatenate([dst, n + lax.rem(ar, 128)])
  idx_t = jnp.stack([src.reshape(_NW, n_chunk, _CH),
                     dst.reshape(_NW, n_chunk, _CH)], axis=2)

  agg = _make_agg(n_acc, n_chunk, d)
  deg = _make_deg(n_acc, n_chunk)

  dp = deg(idx_t)
  g1 = _mm(x, W1)
  p = agg(g1, idx_t)
  g2 = _combine_mm(p, dp, b1.reshape(1, d), W2, n)
  q = agg(g2, idx_t)
  return _final(q, dp, b2.reshape(1, d), n)


# speedup vs baseline: 13.1444x; 1.2151x over previous
import jax
import jax.numpy as jnp
from jax import lax
from jax.experimental import pallas as pl
from jax.experimental.pallas import tpu as pltpu
from jax.experimental.pallas import tpu_sc as plsc

_NC = 2
_NS = 16
_NW = _NC * _NS
_CH = 128
_NBUF = 2
_DEGW = 16


_NIDX = 6


def _make_agg(n_acc, n_chunk, d):
  mesh = plsc.VectorSubcoreMesh(core_axis_name="c", subcore_axis_name="s")
  rows_per_tile = n_acc // _NS
  out_type = jax.ShapeDtypeStruct((_NC, n_acc, d), jnp.float32)

  scratch = [
      pltpu.VMEM_SHARED((n_acc, d), jnp.float32),
      pltpu.VMEM((_NBUF, _CH, d), jnp.float32),
      pltpu.VMEM((_NIDX, 2, _CH), jnp.int32),
      pltpu.SemaphoreType.DMA((_NBUF,)),
      pltpu.SemaphoreType.DMA((_NIDX,)),
  ]

  def body(g_h, idx_h, part_h, acc, rows, islot, gsem, isem):
    c = lax.axis_index("c")
    s = lax.axis_index("s")
    w = c * _NS + s

    zv = jnp.zeros((16,), jnp.float32)
    ncol = d // 16

    def zrow(i, _):
      rows[0, i // ncol, pl.ds(lax.rem(i, ncol) * 16, 16)] = zv
      return 0
    lax.fori_loop(0, _CH * ncol, zrow, 0)

    base = s * rows_per_tile
    for j in range(rows_per_tile // _CH):
      pltpu.sync_copy(rows.at[0], acc.at[pl.ds(base + j * _CH, _CH)])
    plsc.subcore_barrier()

    def stage_idx(k):
      pltpu.async_copy(idx_h.at[w, lax.rem(k, n_chunk)],
                       islot.at[lax.rem(k, _NIDX)],
                       isem.at[lax.rem(k, _NIDX)])

    def wait_idx(k):
      pltpu.make_async_copy(idx_h.at[w, lax.rem(k, n_chunk)],
                            islot.at[lax.rem(k, _NIDX)],
                            isem.at[lax.rem(k, _NIDX)]).wait()

    def issue_gather(k):
      pltpu.async_copy(g_h.at[islot.at[lax.rem(k, _NIDX), 0]],
                       rows.at[lax.rem(k, _NBUF)],
                       gsem.at[lax.rem(k, _NBUF)])

    def wait_gather(k):
      pltpu.make_async_copy(g_h.at[islot.at[lax.rem(k, _NIDX), 0]],
                            rows.at[lax.rem(k, _NBUF)],
                            gsem.at[lax.rem(k, _NBUF)]).wait()

    for j in range(min(_NIDX, n_chunk)):
      stage_idx(j)
    for j in range(min(_NBUF, n_chunk)):
      wait_idx(j)
      issue_gather(j)

    def step(k, _):
      wait_gather(k)
      dsti = islot.at[lax.rem(k, _NIDX), 1]
      pltpu.sync_copy(rows.at[lax.rem(k, _NBUF)], acc.at[dsti], add=True)

      @pl.when(k + _NBUF < n_chunk)
      def _():
        wait_idx(k + _NBUF)
        issue_gather(k + _NBUF)

      @pl.when(k + _NIDX < n_chunk)
      def _():
        stage_idx(k + _NIDX)
      return 0
    lax.fori_loop(0, n_chunk, step, 0)

    plsc.subcore_barrier()
    sl = pl.ds(base, rows_per_tile)
    pltpu.sync_copy(acc.at[sl], part_h.at[c, sl])

  return pl.kernel(body, out_type=out_type, mesh=mesh, scratch_types=scratch)


def _make_deg(n_acc, n_chunk):
  mesh = plsc.VectorSubcoreMesh(core_axis_name="c", subcore_axis_name="s")
  rows_per_tile = n_acc // _NS
  out_type = jax.ShapeDtypeStruct((_NC, n_acc, _DEGW), jnp.float32)

  scratch = [
      pltpu.VMEM_SHARED((n_acc, _DEGW), jnp.float32),
      pltpu.VMEM((_CH, _DEGW), jnp.float32),
      pltpu.VMEM((_CH, _DEGW), jnp.float32),
      pltpu.VMEM((_NIDX, 2, _CH), jnp.int32),
      pltpu.SemaphoreType.DMA((_NIDX,)),
  ]

  def body(idx_h, degp_h, deg_sh, zdeg, ones_v, islot, isem):
    c = lax.axis_index("c")
    s = lax.axis_index("s")
    w = c * _NS + s

    zv = jnp.zeros((16,), jnp.float32)
    ov = jnp.ones((16,), jnp.float32)
    ncol = _DEGW // 16

    def frow(i, _):
      r = i // ncol
      q = pl.ds(lax.rem(i, ncol) * 16, 16)
      zdeg[r, q] = zv
      ones_v[r, q] = ov
      return 0
    lax.fori_loop(0, _CH * ncol, frow, 0)

    base = s * rows_per_tile
    for j in range(rows_per_tile // _CH):
      pltpu.sync_copy(zdeg, deg_sh.at[pl.ds(base + j * _CH, _CH)])
    plsc.subcore_barrier()

    def stage_idx(k):
      pltpu.async_copy(idx_h.at[w, lax.rem(k, n_chunk)],
                       islot.at[lax.rem(k, _NIDX)],
                       isem.at[lax.rem(k, _NIDX)])

    def wait_idx(k):
      pltpu.make_async_copy(idx_h.at[w, lax.rem(k, n_chunk)],
                            islot.at[lax.rem(k, _NIDX)],
                            isem.at[lax.rem(k, _NIDX)]).wait()

    for j in range(min(_NIDX, n_chunk)):
      stage_idx(j)

    def step(k, _):
      wait_idx(k)
      pltpu.sync_copy(ones_v, deg_sh.at[islot.at[lax.rem(k, _NIDX), 1]],
                      add=True)

      @pl.when(k + _NIDX < n_chunk)
      def _():
        stage_idx(k + _NIDX)
      return 0
    lax.fori_loop(0, n_chunk, step, 0)

    plsc.subcore_barrier()
    sl = pl.ds(base, rows_per_tile)
    pltpu.sync_copy(deg_sh.at[sl], degp_h.at[c, sl])

  return pl.kernel(body, out_type=out_type, mesh=mesh, scratch_types=scratch,
                   compiler_params=pltpu.CompilerParams(
                       use_tc_tiling_on_sc=False))


def _mm(x, w):
  n, d = x.shape
  blk = 1000

  def body(x_ref, w_ref, o_ref):
    o_ref[...] = jnp.dot(x_ref[...], w_ref[...],
                         preferred_element_type=jnp.float32)

  return pl.pallas_call(
      body,
      grid=(n // blk,),
      in_specs=[pl.BlockSpec((blk, d), lambda i: (i, 0)),
                pl.BlockSpec((d, d), lambda i: (0, 0))],
      out_specs=pl.BlockSpec((blk, d), lambda i: (i, 0)),
      out_shape=jax.ShapeDtypeStruct((n, d), jnp.float32),
  )(x, w)


def _combine_mm(p, dp, b, w, n):
  d = p.shape[-1]
  blk = 1000

  def body(p_ref, dp_ref, b_ref, w_ref, o_ref):
    pa = p_ref[0] + p_ref[1]
    dg = dp_ref[0, :, 0:1] + dp_ref[1, :, 0:1]
    dinv = 1.0 / jnp.maximum(dg, 1.0)
    h = jnp.maximum(pa * dinv + b_ref[...], 0.0)
    o_ref[...] = jnp.dot(h, w_ref[...], preferred_element_type=jnp.float32)

  return pl.pallas_call(
      body,
      grid=(n // blk,),
      in_specs=[pl.BlockSpec((2, blk, d), lambda i: (0, i, 0)),
                pl.BlockSpec((2, blk, _DEGW), lambda i: (0, i, 0)),
                pl.BlockSpec((1, d), lambda i: (0, 0)),
                pl.BlockSpec((d, d), lambda i: (0, 0))],
      out_specs=pl.BlockSpec((blk, d), lambda i: (i, 0)),
      out_shape=jax.ShapeDtypeStruct((n, d), jnp.float32),
  )(p, dp, b, w)


def _final(q, dp, b, n):
  d = q.shape[-1]
  blk = 1000

  def body(q_ref, dp_ref, b_ref, o_ref):
    qa = q_ref[0] + q_ref[1]
    dg = dp_ref[0, :, 0:1] + dp_ref[1, :, 0:1]
    dinv = 1.0 / jnp.maximum(dg, 1.0)
    o_ref[...] = qa * dinv + b_ref[...]

  return pl.pallas_call(
      body,
      grid=(n // blk,),
      in_specs=[pl.BlockSpec((2, blk, d), lambda i: (0, i, 0)),
                pl.BlockSpec((2, blk, _DEGW), lambda i: (0, i, 0)),
                pl.BlockSpec((1, d), lambda i: (0, 0))],
      out_specs=pl.BlockSpec((blk, d), lambda i: (i, 0)),
      out_shape=jax.ShapeDtypeStruct((n, d), jnp.float32),
  )(q, dp, b)


def kernel(x, edge_index, W1, b1, W2, b2):
  n, d = x.shape
  e = edge_index.shape[1]
  n_acc = -(-(n + 128) // (_NS * 8)) * (_NS * 8)
  n_chunk = -(-e // (_NW * _CH))
  e_pad = n_chunk * _NW * _CH

  src = edge_index[0]
  dst = edge_index[1]
  pad = e_pad - e
  if pad:
    ar = jnp.arange(pad, dtype=jnp.int32)
    src = jnp.concatenate([src, (ar * 131) % n])
    dst = jnp.concatenate([dst, n + lax.rem(ar, 128)])
  idx_t = jnp.stack([src.reshape(_NW, n_chunk, _CH),
                     dst.reshape(_NW, n_chunk, _CH)], axis=2)

  agg = _make_agg(n_acc, n_chunk, d)
  deg = _make_deg(n_acc, n_chunk)

  dp = deg(idx_t)
  g1 = _mm(x, W1)
  p = agg(g1, idx_t)
  g2 = _combine_mm(p, dp, b1.reshape(1, d), W2, n)
  q = agg(g2, idx_t)
  return _final(q, dp, b2.reshape(1, d), n)
